# baseline (speedup 1.0000x reference)
import functools

import jax
import jax.numpy as jnp
from jax import lax
from jax.experimental import pallas as pl
from jax.experimental.pallas import tpu as pltpu
from jax.experimental.pallas import tpu_sc as plsc

N = 10000
D = 256
E = 160000

NC = 2
NS = 16
L = 16
NW = NC * NS

HALF = N // 2
EPC = E // NW
EPA = E // NS
PIECE = 2000
DUMP = HALF
ACC_ROWS = HALF + 8
CHUNK = 48
CBUF = ((EPA + CHUNK - 1) // CHUNK) * CHUNK + CHUNK

RPT = 313
RPT_LAST = HALF - (NS - 1) * RPT

_mesh = plsc.VectorSubcoreMesh(
    core_axis_name="c", subcore_axis_name="s", num_cores=NC, num_subcores=NS)
_sc_params = pltpu.CompilerParams(
    needs_layout_passes=False, use_tc_tiling_on_sc=False)


def _row_chunks(n, step=CHUNK):
    out = []
    while n > 0:
        out.append(min(step, n))
        n -= step
    return out


@functools.partial(
    pl.kernel,
    out_type=jax.ShapeDtypeStruct((NW, N), jnp.float32),
    mesh=_mesh,
    compiler_params=_sc_params,
)
def _sc_count(ei_hbm, out_hbm):
    c = lax.axis_index("c")
    s = lax.axis_index("s")
    wid = s * NC + c

    def body(dst_v, cnt_v):
        pltpu.sync_copy(ei_hbm.at[1, pl.ds(wid * EPC, EPC)], dst_v)

        zeros = jnp.zeros((L,), jnp.float32)

        def zbody(i, _):
            cnt_v[pl.ds(i * L, L)] = zeros
            return 0

        lax.fori_loop(0, (N + L) // L, zbody, 0)

        ones = jnp.ones((L,), jnp.float32)
        lane = lax.iota(jnp.int32, L)

        def cbody(i, _):
            base = i * L
            m = (base + lane) < EPC
            idx = dst_v[pl.ds(base, L)]
            idx = jnp.where(m, idx, N)
            plsc.addupdate_scatter(cnt_v, [idx], ones)
            return 0

        lax.fori_loop(0, (EPC + L - 1) // L, cbody, 0)
        pltpu.sync_copy(cnt_v.at[pl.ds(0, N)], out_hbm.at[wid])

    pl.run_scoped(
        body,
        pltpu.VMEM((EPC,), jnp.int32),
        pltpu.VMEM((N + L,), jnp.float32),
    )


def _tc_fuse_body(q_ref, o_ref, cnt_ref, wf_ref, bf_ref, wg_ref, y_ref):
    fused = (q_ref[...] @ wf_ref[0:D, :]
             + o_ref[...] @ wf_ref[D:2 * D, :]
             + bf_ref[...])
    x = fused @ wg_ref[...]
    deg = 1.0 + jnp.sum(cnt_ref[...], axis=0)
    dinv = lax.rsqrt(deg)
    y_ref[...] = x * dinv[:, None]


def _tc_fuse(q, obj, cnts, w_fuse, b_fuse, w_gcn):
    return pl.pallas_call(
        _tc_fuse_body,
        out_shape=jax.ShapeDtypeStruct((N, D), jnp.float32),
    )(q, obj, cnts, w_fuse, b_fuse, w_gcn)


@functools.partial(
    pl.kernel,
    out_type=jax.ShapeDtypeStruct((NC, HALF, D), jnp.float32),
    mesh=_mesh,
    compiler_params=_sc_params,
    scratch_types=[
        pltpu.VMEM_SHARED((ACC_ROWS, D), jnp.float32),
        pltpu.SemaphoreType.DMA,
        pltpu.SemaphoreType.DMA,
        pltpu.SemaphoreType.DMA,
        pltpu.SemaphoreType.DMA,
    ],
)
def _sc_agg(ei_hbm, y_hbm, acc_hbm, acc_sh, semA, semB, semSA, semSB):
    c = lax.axis_index("c")
    s = lax.axis_index("s")
    lo = c * HALF
    hi = lo + HALF
    r0 = s * RPT

    def body(ei_src, ei_dst, csrc, cdstf, rowsA, rowsB):
        rows = rowsA
        def _init(nrows):
            def f():
                off = 0
                for nr in _row_chunks(nrows):
                    pltpu.sync_copy(y_hbm.at[pl.ds(lo + r0 + off, nr)],
                                    rows.at[pl.ds(0, nr)])
                    pltpu.sync_copy(rows.at[pl.ds(0, nr)],
                                    acc_sh.at[pl.ds(r0 + off, nr)])
                    off += nr
            return f

        pl.when(s < NS - 1)(_init(RPT))
        pl.when(s == NS - 1)(_init(RPT_LAST))

        zero_i = jnp.zeros((L,), jnp.int32)
        dump_i = jnp.full((L,), DUMP, jnp.int32)

        def pbody(i, _):
            csrc[pl.ds(i * L, L)] = zero_i
            cdstf[pl.ds(i * L, L)] = dump_i
            return 0

        lax.fori_loop(0, CBUF // L, pbody, 0)

        def piece(p, cnt):
            e0 = s * EPA + p * PIECE
            pltpu.sync_copy(ei_hbm.at[0, pl.ds(e0, PIECE)], ei_src)
            pltpu.sync_copy(ei_hbm.at[1, pl.ds(e0, PIECE)], ei_dst)

            def cbody(i, cnt):
                base = i * L
                d16 = ei_dst[pl.ds(base, L)]
                s16 = ei_src[pl.ds(base, L)]
                m = (d16 >= lo) & (d16 < hi)
                plsc.store_compressed(csrc.at[pl.ds(cnt, L)], s16, mask=m)
                plsc.store_compressed(cdstf.at[pl.ds(cnt, L)], d16 - lo,
                                      mask=m)
                return cnt + jnp.sum(m.astype(jnp.int32))

            return lax.fori_loop(0, PIECE // L, cbody, cnt)

        cnt = lax.fori_loop(0, EPA // PIECE, piece, 0)

        plsc.subcore_barrier()

        nch = (cnt + CHUNK - 1) // CHUNK

        def fire_g(j, buf, sem):
            pltpu.async_copy(y_hbm.at[csrc.at[pl.ds(j * CHUNK, CHUNK)]],
                             buf, sem)

        def wait_g(buf, sem):
            pltpu.make_async_copy(y_hbm.at[pl.ds(0, CHUNK)], buf, sem).wait()

        def fire_s(j, buf, sem):
            pltpu.async_copy(
                buf, acc_sh.at[cdstf.at[pl.ds(j * CHUNK, CHUNK)]], sem,
                add=True)

        def wait_s(buf, sem):
            pltpu.make_async_copy(buf, acc_sh.at[pl.ds(0, CHUNK)],
                                  sem).wait()

        pl.when(nch > 0)(lambda: fire_g(0, rowsA, semA))
        pl.when(nch > 1)(lambda: fire_g(1, rowsB, semB))

        def pbody(jj, _):
            a = 2 * jj
            b = a + 1
            wait_g(rowsA, semA)
            fire_s(a, rowsA, semSA)

            def do_b():
                wait_g(rowsB, semB)
                fire_s(b, rowsB, semSB)

            pl.when(b < nch)(do_b)

            def next_a():
                wait_s(rowsA, semSA)
                fire_g(a + 2, rowsA, semA)

            pl.when(a + 2 < nch)(next_a)

            def next_b():
                wait_s(rowsB, semSB)
                fire_g(b + 2, rowsB, semB)

            pl.when(b + 2 < nch)(next_b)
            return 0

        lax.fori_loop(0, (nch + 1) // 2, pbody, 0)

        pl.when(nch >= 1)(lambda: wait_s(rowsA, semSA))
        pl.when(nch >= 2)(lambda: wait_s(rowsB, semSB))

        plsc.subcore_barrier()

        def _wb(nrows):
            def f():
                pltpu.sync_copy(acc_sh.at[pl.ds(r0, nrows)],
                                acc_hbm.at[c, pl.ds(r0, nrows)])
            return f

        pl.when(s < NS - 1)(_wb(RPT))
        pl.when(s == NS - 1)(_wb(RPT_LAST))

    pl.run_scoped(
        body,
        pltpu.VMEM((PIECE,), jnp.int32),
        pltpu.VMEM((PIECE,), jnp.int32),
        pltpu.VMEM((CBUF,), jnp.int32),
        pltpu.VMEM((CBUF,), jnp.int32),
        pltpu.VMEM((CHUNK, D), jnp.float32),
        pltpu.VMEM((CHUNK, D), jnp.float32),
    )


def _tc_fin_body(acc_ref, cnt_ref, bg_ref, out_ref):
    deg = 1.0 + jnp.sum(cnt_ref[...], axis=0)
    dinv = lax.rsqrt(deg)
    acc = acc_ref[...].reshape(N, D)
    out_ref[...] = acc * dinv[:, None] + bg_ref[...]


def _tc_fin(acc, cnts, b_gcn):
    return pl.pallas_call(
        _tc_fin_body,
        out_shape=jax.ShapeDtypeStruct((N, D), jnp.float32),
    )(acc, cnts, b_gcn)


def kernel(question_embedding, object_features_list, bounding_boxes,
           batch_size, num_obj, edge_index, batch,
           W_fuse, b_fuse, W_gcn, b_gcn):
    cnts = _sc_count(edge_index)
    y = _tc_fuse(question_embedding, object_features_list, cnts,
                 W_fuse, b_fuse.reshape(1, D), W_gcn)
    acc = _sc_agg(edge_index, y)
    out = _tc_fin(acc, cnts, b_gcn.reshape(1, D))
    return out


# speedup vs baseline: 1.1180x; 1.1180x over previous
import functools

import jax
import jax.numpy as jnp
from jax import lax
from jax.experimental import pallas as pl
from jax.experimental.pallas import tpu as pltpu
from jax.experimental.pallas import tpu_sc as plsc

N = 10000
D = 256
E = 160000

NC = 2
NS = 16
L = 16
NW = NC * NS

HALF = N // 2
EPC = E // NW
EPA = E // NS
PIECE = 2000
DUMP = HALF
ACC_ROWS = HALF + 8
CHUNK = 48
CBUF = ((EPA + CHUNK - 1) // CHUNK) * CHUNK + CHUNK

RPT = 313
RPT_LAST = HALF - (NS - 1) * RPT

_mesh = plsc.VectorSubcoreMesh(
    core_axis_name="c", subcore_axis_name="s", num_cores=NC, num_subcores=NS)
_sc_params = pltpu.CompilerParams(
    needs_layout_passes=False, use_tc_tiling_on_sc=False)


def _row_chunks(n, step=CHUNK):
    out = []
    while n > 0:
        out.append(min(step, n))
        n -= step
    return out


@functools.partial(
    pl.kernel,
    out_type=jax.ShapeDtypeStruct((NW, N), jnp.float32),
    mesh=_mesh,
    compiler_params=_sc_params,
)
def _sc_count(ei_hbm, out_hbm):
    c = lax.axis_index("c")
    s = lax.axis_index("s")
    wid = s * NC + c

    def body(dst_v, cnt_v):
        pltpu.sync_copy(ei_hbm.at[1, pl.ds(wid * EPC, EPC)], dst_v)

        zeros = jnp.zeros((L,), jnp.float32)

        def zbody(i, _):
            cnt_v[pl.ds(i * L, L)] = zeros
            return 0

        lax.fori_loop(0, (N + L) // L, zbody, 0)

        ones = jnp.ones((L,), jnp.float32)
        lane = lax.iota(jnp.int32, L)

        def cbody(i, _):
            base = i * L
            m = (base + lane) < EPC
            idx = dst_v[pl.ds(base, L)]
            idx = jnp.where(m, idx, N)
            plsc.addupdate_scatter(cnt_v, [idx], ones)
            return 0

        lax.fori_loop(0, (EPC + L - 1) // L, cbody, 0)
        pltpu.sync_copy(cnt_v.at[pl.ds(0, N)], out_hbm.at[wid])

    pl.run_scoped(
        body,
        pltpu.VMEM((EPC,), jnp.int32),
        pltpu.VMEM((N + L,), jnp.float32),
    )


def _tc_fuse_body(q_ref, o_ref, cnt_ref, wf_ref, bf_ref, wg_ref,
                  yl_ref, yr_ref):
    fused = (q_ref[...] @ wf_ref[0:D, :]
             + o_ref[...] @ wf_ref[D:2 * D, :]
             + bf_ref[...])
    x = fused @ wg_ref[...]
    deg = 1.0 + jnp.sum(cnt_ref[...], axis=0)
    dinv = lax.rsqrt(deg)
    y = x * dinv[:, None]
    yl_ref[...] = y[:, :D // 2]
    yr_ref[...] = y[:, D // 2:]


def _tc_fuse(q, obj, cnts, w_fuse, b_fuse, w_gcn):
    return pl.pallas_call(
        _tc_fuse_body,
        out_shape=[jax.ShapeDtypeStruct((N, D // 2), jnp.float32),
                   jax.ShapeDtypeStruct((N, D // 2), jnp.float32)],
    )(q, obj, cnts, w_fuse, b_fuse, w_gcn)


@functools.partial(
    pl.kernel,
    out_type=[jax.ShapeDtypeStruct((NC, HALF, D // 2), jnp.float32),
              jax.ShapeDtypeStruct((NC, HALF, D // 2), jnp.float32)],
    mesh=_mesh,
    compiler_params=_sc_params,
    scratch_types=[
        pltpu.VMEM_SHARED((ACC_ROWS, D // 2), jnp.float32),
        pltpu.VMEM_SHARED((ACC_ROWS, D // 2), jnp.float32),
        pltpu.SemaphoreType.DMA,
        pltpu.SemaphoreType.DMA,
        pltpu.SemaphoreType.DMA,
    ],
)
def _sc_agg(ei_hbm, yl_hbm, yr_hbm, accl_hbm, accr_hbm,
            accl_sh, accr_sh, semA, semB, semS):
    c = lax.axis_index("c")
    s = lax.axis_index("s")
    lo = c * HALF
    hi = lo + HALF
    r0 = s * RPT

    def body(ei_src, ei_dst, csrc, cdstf, rowsAL, rowsAR, rowsBL, rowsBR):
        def _init(nrows):
            def f():
                off = 0
                for nr in _row_chunks(nrows):
                    pltpu.sync_copy(yl_hbm.at[pl.ds(lo + r0 + off, nr)],
                                    rowsAL.at[pl.ds(0, nr)])
                    pltpu.sync_copy(yr_hbm.at[pl.ds(lo + r0 + off, nr)],
                                    rowsAR.at[pl.ds(0, nr)])
                    pltpu.sync_copy(rowsAL.at[pl.ds(0, nr)],
                                    accl_sh.at[pl.ds(r0 + off, nr)])
                    pltpu.sync_copy(rowsAR.at[pl.ds(0, nr)],
                                    accr_sh.at[pl.ds(r0 + off, nr)])
                    off += nr
            return f

        pl.when(s < NS - 1)(_init(RPT))
        pl.when(s == NS - 1)(_init(RPT_LAST))

        zero_i = jnp.zeros((L,), jnp.int32)
        dump_i = jnp.full((L,), DUMP, jnp.int32)

        def pbody(i, _):
            csrc[pl.ds(i * L, L)] = zero_i
            cdstf[pl.ds(i * L, L)] = dump_i
            return 0

        lax.fori_loop(0, CBUF // L, pbody, 0)

        def piece(p, cnt):
            e0 = s * EPA + p * PIECE
            pltpu.sync_copy(ei_hbm.at[0, pl.ds(e0, PIECE)], ei_src)
            pltpu.sync_copy(ei_hbm.at[1, pl.ds(e0, PIECE)], ei_dst)

            def cbody(i, cnt):
                base = i * L
                d16 = ei_dst[pl.ds(base, L)]
                s16 = ei_src[pl.ds(base, L)]
                m = (d16 >= lo) & (d16 < hi)
                plsc.store_compressed(csrc.at[pl.ds(cnt, L)], s16, mask=m)
                plsc.store_compressed(cdstf.at[pl.ds(cnt, L)], d16 - lo,
                                      mask=m)
                return cnt + jnp.sum(m.astype(jnp.int32))

            return lax.fori_loop(0, PIECE // L, cbody, cnt)

        cnt = lax.fori_loop(0, EPA // PIECE, piece, 0)

        plsc.subcore_barrier()

        nch = (cnt + CHUNK - 1) // CHUNK

        def fire_g(j, bufL, bufR, sem):
            idx = csrc.at[pl.ds(j * CHUNK, CHUNK)]
            pltpu.async_copy(yl_hbm.at[idx], bufL, sem)
            pltpu.async_copy(yr_hbm.at[idx], bufR, sem)

        def wait_g(bufL, bufR, sem):
            pltpu.make_async_copy(yl_hbm.at[pl.ds(0, CHUNK)], bufL,
                                  sem).wait()
            pltpu.make_async_copy(yr_hbm.at[pl.ds(0, CHUNK)], bufR,
                                  sem).wait()

        def scat(j, bufL, bufR):
            idx = cdstf.at[pl.ds(j * CHUNK, CHUNK)]
            dl = pltpu.async_copy(bufL, accl_sh.at[idx], semS, add=True)
            dr = pltpu.async_copy(bufR, accr_sh.at[idx], semS, add=True)
            dl.wait()
            dr.wait()

        pl.when(nch > 0)(lambda: fire_g(0, rowsAL, rowsAR, semA))

        def pbody(jj, _):
            a = 2 * jj
            b = a + 1
            wait_g(rowsAL, rowsAR, semA)
            pl.when(b < nch)(lambda: fire_g(b, rowsBL, rowsBR, semB))
            scat(a, rowsAL, rowsAR)

            def do_b():
                wait_g(rowsBL, rowsBR, semB)
                pl.when(a + 2 < nch)(
                    lambda: fire_g(a + 2, rowsAL, rowsAR, semA))
                scat(b, rowsBL, rowsBR)

            pl.when(b < nch)(do_b)
            return 0

        lax.fori_loop(0, (nch + 1) // 2, pbody, 0)

        plsc.subcore_barrier()

        def _wb(nrows):
            def f():
                pltpu.sync_copy(accl_sh.at[pl.ds(r0, nrows)],
                                accl_hbm.at[c, pl.ds(r0, nrows)])
                pltpu.sync_copy(accr_sh.at[pl.ds(r0, nrows)],
                                accr_hbm.at[c, pl.ds(r0, nrows)])
            return f

        pl.when(s < NS - 1)(_wb(RPT))
        pl.when(s == NS - 1)(_wb(RPT_LAST))

    pl.run_scoped(
        body,
        pltpu.VMEM((PIECE,), jnp.int32),
        pltpu.VMEM((PIECE,), jnp.int32),
        pltpu.VMEM((CBUF,), jnp.int32),
        pltpu.VMEM((CBUF,), jnp.int32),
        pltpu.VMEM((CHUNK, D // 2), jnp.float32),
        pltpu.VMEM((CHUNK, D // 2), jnp.float32),
        pltpu.VMEM((CHUNK, D // 2), jnp.float32),
        pltpu.VMEM((CHUNK, D // 2), jnp.float32),
    )


def _tc_fin_body(accl_ref, accr_ref, cnt_ref, bg_ref, out_ref):
    deg = 1.0 + jnp.sum(cnt_ref[...], axis=0)
    dinv = lax.rsqrt(deg)
    acc = jnp.concatenate([accl_ref[...].reshape(N, D // 2),
                           accr_ref[...].reshape(N, D // 2)], axis=1)
    out_ref[...] = acc * dinv[:, None] + bg_ref[...]


def _tc_fin(accl, accr, cnts, b_gcn):
    return pl.pallas_call(
        _tc_fin_body,
        out_shape=jax.ShapeDtypeStruct((N, D), jnp.float32),
    )(accl, accr, cnts, b_gcn)


def kernel(question_embedding, object_features_list, bounding_boxes,
           batch_size, num_obj, edge_index, batch,
           W_fuse, b_fuse, W_gcn, b_gcn):
    cnts = _sc_count(edge_index)
    yl, yr = _tc_fuse(question_embedding, object_features_list, cnts,
                      W_fuse, b_fuse.reshape(1, D), W_gcn)
    accl, accr = _sc_agg(edge_index, yl, yr)
    out = _tc_fin(accl, accr, cnts, b_gcn.reshape(1, D))
    return out


# speedup vs baseline: 1.1306x; 1.0113x over previous
import functools

import jax
import jax.numpy as jnp
from jax import lax
from jax.experimental import pallas as pl
from jax.experimental.pallas import tpu as pltpu
from jax.experimental.pallas import tpu_sc as plsc

N = 10000
D = 256
E = 160000

NC = 2
NS = 16
L = 16
NW = NC * NS

HALF = N // 2
EPC = E // NW
EPA = E // NS
PIECE = 2000
DUMP = HALF
ACC_ROWS = HALF + 8
CHUNK = 48
CBUF = ((EPA + CHUNK - 1) // CHUNK) * CHUNK + CHUNK

RPT = 313
RPT_LAST = HALF - (NS - 1) * RPT

_mesh = plsc.VectorSubcoreMesh(
    core_axis_name="c", subcore_axis_name="s", num_cores=NC, num_subcores=NS)
_sc_params = pltpu.CompilerParams(
    needs_layout_passes=False, use_tc_tiling_on_sc=False)


def _row_chunks(n, step=CHUNK):
    out = []
    while n > 0:
        out.append(min(step, n))
        n -= step
    return out


@functools.partial(
    pl.kernel,
    out_type=jax.ShapeDtypeStruct((NW, N), jnp.float32),
    mesh=_mesh,
    compiler_params=_sc_params,
)
def _sc_count(ei_hbm, out_hbm):
    c = lax.axis_index("c")
    s = lax.axis_index("s")
    wid = s * NC + c

    def body(dst_v, cnt_v):
        pltpu.sync_copy(ei_hbm.at[1, pl.ds(wid * EPC, EPC)], dst_v)

        zeros = jnp.zeros((L,), jnp.float32)

        def zbody(i, _):
            cnt_v[pl.ds(i * L, L)] = zeros
            return 0

        lax.fori_loop(0, (N + L) // L, zbody, 0)

        ones = jnp.ones((L,), jnp.float32)
        lane = lax.iota(jnp.int32, L)

        def cbody(i, _):
            base = i * L
            m = (base + lane) < EPC
            idx = dst_v[pl.ds(base, L)]
            idx = jnp.where(m, idx, N)
            plsc.addupdate_scatter(cnt_v, [idx], ones)
            return 0

        lax.fori_loop(0, (EPC + L - 1) // L, cbody, 0)
        pltpu.sync_copy(cnt_v.at[pl.ds(0, N)], out_hbm.at[wid])

    pl.run_scoped(
        body,
        pltpu.VMEM((EPC,), jnp.int32),
        pltpu.VMEM((N + L,), jnp.float32),
    )


def _tc_fuse_body(q_ref, o_ref, cnt_ref, wf_ref, bf_ref, wg_ref,
                  yl_ref, yr_ref):
    fused = (q_ref[...] @ wf_ref[0:D, :]
             + o_ref[...] @ wf_ref[D:2 * D, :]
             + bf_ref[...])
    x = fused @ wg_ref[...]
    deg = 1.0 + jnp.sum(cnt_ref[...], axis=0)
    dinv = lax.rsqrt(deg)
    y = x * dinv[:, None]
    yl_ref[...] = y[:, :D // 2]
    yr_ref[...] = y[:, D // 2:]


def _tc_fuse(q, obj, cnts, w_fuse, b_fuse, w_gcn):
    return pl.pallas_call(
        _tc_fuse_body,
        out_shape=[jax.ShapeDtypeStruct((N, D // 2), jnp.float32),
                   jax.ShapeDtypeStruct((N, D // 2), jnp.float32)],
    )(q, obj, cnts, w_fuse, b_fuse, w_gcn)


@functools.partial(
    pl.kernel,
    out_type=[jax.ShapeDtypeStruct((NC, HALF, D // 2), jnp.float32),
              jax.ShapeDtypeStruct((NC, HALF, D // 2), jnp.float32)],
    mesh=_mesh,
    compiler_params=_sc_params,
    scratch_types=[
        pltpu.VMEM_SHARED((ACC_ROWS, D // 2), jnp.float32),
        pltpu.VMEM_SHARED((ACC_ROWS, D // 2), jnp.float32),
        pltpu.SemaphoreType.DMA,
        pltpu.SemaphoreType.DMA,
        pltpu.SemaphoreType.DMA,
    ],
)
def _sc_agg(ei_hbm, yl_hbm, yr_hbm, accl_hbm, accr_hbm,
            accl_sh, accr_sh, semA, semB, semS):
    c = lax.axis_index("c")
    s = lax.axis_index("s")
    lo = c * HALF
    hi = lo + HALF
    r0 = s * RPT

    def body(ei_src, ei_dst, csrc, cdstf, rowsAL, rowsAR, rowsBL, rowsBR):
        def _init(nrows):
            def f():
                off = 0
                for nr in _row_chunks(nrows):
                    pltpu.sync_copy(yl_hbm.at[pl.ds(lo + r0 + off, nr)],
                                    rowsAL.at[pl.ds(0, nr)])
                    pltpu.sync_copy(yr_hbm.at[pl.ds(lo + r0 + off, nr)],
                                    rowsAR.at[pl.ds(0, nr)])
                    pltpu.sync_copy(rowsAL.at[pl.ds(0, nr)],
                                    accl_sh.at[pl.ds(r0 + off, nr)])
                    pltpu.sync_copy(rowsAR.at[pl.ds(0, nr)],
                                    accr_sh.at[pl.ds(r0 + off, nr)])
                    off += nr
            return f

        pl.when(s < NS - 1)(_init(RPT))
        pl.when(s == NS - 1)(_init(RPT_LAST))

        zero_i = jnp.zeros((L,), jnp.int32)
        dump_i = jnp.full((L,), DUMP, jnp.int32)

        def piece(p, cnt):
            e0 = s * EPA + p * PIECE
            pltpu.sync_copy(ei_hbm.at[0, pl.ds(e0, PIECE)], ei_src)
            pltpu.sync_copy(ei_hbm.at[1, pl.ds(e0, PIECE)], ei_dst)

            def cbody(i, cnt):
                base = i * L
                d16 = ei_dst[pl.ds(base, L)]
                s16 = ei_src[pl.ds(base, L)]
                m = (d16 >= lo) & (d16 < hi)
                plsc.store_compressed(csrc.at[pl.ds(cnt, L)], s16, mask=m)
                plsc.store_compressed(cdstf.at[pl.ds(cnt, L)], d16 - lo,
                                      mask=m)
                return cnt + jnp.sum(m.astype(jnp.int32))

            return lax.fori_loop(0, PIECE // L, cbody, cnt)

        cnt = lax.fori_loop(0, EPA // PIECE, piece, 0)

        full_m = jnp.ones((L,), jnp.bool_)

        def tail(k, _):
            plsc.store_compressed(csrc.at[pl.ds(cnt + k * L, L)], zero_i,
                                  mask=full_m)
            plsc.store_compressed(cdstf.at[pl.ds(cnt + k * L, L)], dump_i,
                                  mask=full_m)
            return 0

        lax.fori_loop(0, CHUNK // L, tail, 0)

        plsc.subcore_barrier()

        nch = (cnt + CHUNK - 1) // CHUNK

        def fire_g(j, bufL, bufR, sem):
            idx = csrc.at[pl.ds(j * CHUNK, CHUNK)]
            pltpu.async_copy(yl_hbm.at[idx], bufL, sem)
            pltpu.async_copy(yr_hbm.at[idx], bufR, sem)

        def wait_g(bufL, bufR, sem):
            pltpu.make_async_copy(yl_hbm.at[pl.ds(0, CHUNK)], bufL,
                                  sem).wait()
            pltpu.make_async_copy(yr_hbm.at[pl.ds(0, CHUNK)], bufR,
                                  sem).wait()

        def scat(j, bufL, bufR):
            idx = cdstf.at[pl.ds(j * CHUNK, CHUNK)]
            dl = pltpu.async_copy(bufL, accl_sh.at[idx], semS, add=True)
            dr = pltpu.async_copy(bufR, accr_sh.at[idx], semS, add=True)
            dl.wait()
            dr.wait()

        pl.when(nch > 0)(lambda: fire_g(0, rowsAL, rowsAR, semA))

        def pbody(jj, _):
            a = 2 * jj
            b = a + 1
            wait_g(rowsAL, rowsAR, semA)
            pl.when(b < nch)(lambda: fire_g(b, rowsBL, rowsBR, semB))
            scat(a, rowsAL, rowsAR)

            def do_b():
                wait_g(rowsBL, rowsBR, semB)
                pl.when(a + 2 < nch)(
                    lambda: fire_g(a + 2, rowsAL, rowsAR, semA))
                scat(b, rowsBL, rowsBR)

            pl.when(b < nch)(do_b)
            return 0

        lax.fori_loop(0, (nch + 1) // 2, pbody, 0)

        plsc.subcore_barrier()

        def _wb(nrows):
            def f():
                pltpu.sync_copy(accl_sh.at[pl.ds(r0, nrows)],
                                accl_hbm.at[c, pl.ds(r0, nrows)])
                pltpu.sync_copy(accr_sh.at[pl.ds(r0, nrows)],
                                accr_hbm.at[c, pl.ds(r0, nrows)])
            return f

        pl.when(s < NS - 1)(_wb(RPT))
        pl.when(s == NS - 1)(_wb(RPT_LAST))

    pl.run_scoped(
        body,
        pltpu.VMEM((PIECE,), jnp.int32),
        pltpu.VMEM((PIECE,), jnp.int32),
        pltpu.VMEM((CBUF,), jnp.int32),
        pltpu.VMEM((CBUF,), jnp.int32),
        pltpu.VMEM((CHUNK, D // 2), jnp.float32),
        pltpu.VMEM((CHUNK, D // 2), jnp.float32),
        pltpu.VMEM((CHUNK, D // 2), jnp.float32),
        pltpu.VMEM((CHUNK, D // 2), jnp.float32),
    )


def _tc_fin_body(accl_ref, accr_ref, cnt_ref, bg_ref, out_ref):
    deg = 1.0 + jnp.sum(cnt_ref[...], axis=0)
    dinv = lax.rsqrt(deg)
    acc = jnp.concatenate([accl_ref[...].reshape(N, D // 2),
                           accr_ref[...].reshape(N, D // 2)], axis=1)
    out_ref[...] = acc * dinv[:, None] + bg_ref[...]


def _tc_fin(accl, accr, cnts, b_gcn):
    return pl.pallas_call(
        _tc_fin_body,
        out_shape=jax.ShapeDtypeStruct((N, D), jnp.float32),
    )(accl, accr, cnts, b_gcn)


def kernel(question_embedding, object_features_list, bounding_boxes,
           batch_size, num_obj, edge_index, batch,
           W_fuse, b_fuse, W_gcn, b_gcn):
    cnts = _sc_count(edge_index)
    yl, yr = _tc_fuse(question_embedding, object_features_list, cnts,
                      W_fuse, b_fuse.reshape(1, D), W_gcn)
    accl, accr = _sc_agg(edge_index, yl, yr)
    out = _tc_fin(accl, accr, cnts, b_gcn.reshape(1, D))
    return out
